# Initial kernel scaffold; baseline (speedup 1.0000x reference)
import jax
import jax.numpy as jnp
from jax.experimental import pallas as pl


def kernel(x, edge_index, action_one_hot, node1_emb, node2_emb, action_prob, W1, b1, W2, b2, Wp, bp, Wf1, bf1, Wf2, bf2):
    raise NotImplementedError("write your pallas kernel here")


# speedup vs baseline: 19.3931x; 19.3931x over previous
import functools

import jax
import jax.numpy as jnp
from jax import lax
from jax.experimental import pallas as pl
from jax.experimental.pallas import tpu as pltpu
from jax.experimental.pallas import tpu_sc as plsc

N_NODES = 10000
D = 128
N_EDGES = 320000
D_PROJ = 64

NC, NS = 2, 16
NW = NC * NS
E_PER_TILE = N_EDGES // NW
CH = 80
NCHUNK = E_PER_TILE // CH
ROWS_PER_TILE = N_NODES // NS
NS_IO = 10
IO_ROWS = N_NODES // NS_IO
DEG_W = 16

BLK = 1000
GRID = N_NODES // BLK


def _deg_body(dst_hbm, ones_hbm, zdeg_hbm, out_hbm, dst_v, ones_v, deg_sh):
    cid = lax.axis_index("c")
    sid = lax.axis_index("s")
    w = cid * NS + sid
    pltpu.sync_copy(dst_hbm.at[w], dst_v)
    pltpu.sync_copy(ones_hbm, ones_v)

    @pl.when(sid < NS_IO)
    def _():
        pltpu.sync_copy(zdeg_hbm, deg_sh.at[pl.ds(sid * IO_ROWS, IO_ROWS)])

    plsc.subcore_barrier()

    def body(j, carry):
        pltpu.sync_copy(ones_v, deg_sh.at[dst_v.at[j]], add=True)
        return carry

    lax.fori_loop(0, NCHUNK, body, 0)
    plsc.subcore_barrier()

    @pl.when(sid < NS_IO)
    def _():
        pltpu.sync_copy(deg_sh.at[pl.ds(sid * IO_ROWS, IO_ROWS)],
                        out_hbm.at[cid].at[pl.ds(sid * IO_ROWS, IO_ROWS)])


@functools.cache
def _sc_mesh():
    return plsc.VectorSubcoreMesh(
        core_axis_name="c", subcore_axis_name="s",
        num_cores=NC, num_subcores=NS)


@functools.cache
def _deg_kernel():
    return pl.kernel(
        _deg_body,
        out_type=jax.ShapeDtypeStruct((NC, N_NODES, DEG_W), jnp.float32),
        mesh=_sc_mesh(),
        compiler_params=pltpu.CompilerParams(use_tc_tiling_on_sc=False),
        scratch_types=[
            pltpu.VMEM((NCHUNK, CH), jnp.int32),
            pltpu.VMEM((CH, DEG_W), jnp.float32),
            pltpu.VMEM_SHARED((N_NODES, DEG_W), jnp.float32),
        ])


def _edge_body(hs_hbm, src_hbm, dst_hbm, zrow_hbm, out_hbm,
               src_v, dst_v, rows_v, acc_sh):
    cid = lax.axis_index("c")
    sid = lax.axis_index("s")
    w = cid * NS + sid
    pltpu.sync_copy(src_hbm.at[w], src_v)
    pltpu.sync_copy(dst_hbm.at[w], dst_v)

    @pl.when(sid < NS_IO)
    def _():
        pltpu.sync_copy(zrow_hbm, acc_sh.at[pl.ds(sid * IO_ROWS, IO_ROWS)])

    plsc.subcore_barrier()

    def body(j, carry):
        pltpu.sync_copy(hs_hbm.at[src_v.at[j]], rows_v)
        pltpu.sync_copy(rows_v, acc_sh.at[dst_v.at[j]], add=True)
        return carry

    lax.fori_loop(0, NCHUNK, body, 0)
    plsc.subcore_barrier()

    @pl.when(sid < NS_IO)
    def _():
        pltpu.sync_copy(acc_sh.at[pl.ds(sid * IO_ROWS, IO_ROWS)],
                        out_hbm.at[cid].at[pl.ds(sid * IO_ROWS, IO_ROWS)])


@functools.cache
def _edge_kernel():
    return pl.kernel(
        _edge_body,
        out_type=jax.ShapeDtypeStruct((NC, N_NODES, D), jnp.float32),
        mesh=_sc_mesh(),
        scratch_types=[
            pltpu.VMEM((NCHUNK, CH), jnp.int32),
            pltpu.VMEM((NCHUNK, CH), jnp.int32),
            pltpu.VMEM((CH, D), jnp.float32),
            pltpu.VMEM_SHARED((N_NODES, D), jnp.float32),
        ])


def _mm_scale_body(x_ref, w_ref, deg_ref, o_ref):
    deg = deg_ref[0, :, 0] + deg_ref[1, :, 0] + 1.0
    dis = lax.rsqrt(deg)
    h = jnp.dot(x_ref[...], w_ref[...], preferred_element_type=jnp.float32)
    o_ref[...] = h * dis[:, None]


def _mm_scale(x, w1, degp):
    return pl.pallas_call(
        _mm_scale_body,
        grid=(GRID,),
        in_specs=[
            pl.BlockSpec((BLK, D), lambda i: (i, 0)),
            pl.BlockSpec((D, D), lambda i: (0, 0)),
            pl.BlockSpec((NC, BLK, DEG_W), lambda i: (0, i, 0)),
        ],
        out_specs=pl.BlockSpec((BLK, D), lambda i: (i, 0)),
        out_shape=jax.ShapeDtypeStruct((N_NODES, D), jnp.float32),
    )(x, w1, degp)


def _layer_body(p_ref, hs_ref, deg_ref, b_ref, w_ref, o_ref):
    deg = deg_ref[0, :, 0] + deg_ref[1, :, 0] + 1.0
    dis = lax.rsqrt(deg)
    agg = p_ref[0] + p_ref[1] + hs_ref[...]
    o1 = jnp.maximum(agg * dis[:, None] + b_ref[...], 0.0)
    h2 = jnp.dot(o1, w_ref[...], preferred_element_type=jnp.float32)
    o_ref[...] = h2 * dis[:, None]


def _layer_mm(p, hs, degp, b, w2):
    return pl.pallas_call(
        _layer_body,
        grid=(GRID,),
        in_specs=[
            pl.BlockSpec((NC, BLK, D), lambda i: (0, i, 0)),
            pl.BlockSpec((BLK, D), lambda i: (i, 0)),
            pl.BlockSpec((NC, BLK, DEG_W), lambda i: (0, i, 0)),
            pl.BlockSpec((1, D), lambda i: (0, 0)),
            pl.BlockSpec((D, D), lambda i: (0, 0)),
        ],
        out_specs=pl.BlockSpec((BLK, D), lambda i: (i, 0)),
        out_shape=jax.ShapeDtypeStruct((N_NODES, D), jnp.float32),
    )(p, hs, degp, b, w2)


def _final_body(p_ref, hs_ref, deg_ref, b2_ref, n1_ref, n2_ref, wp_ref, bp_ref,
                extra_ref, w1g_ref, w1n1_ref, w1n2_ref, w1e_ref, bf1_ref,
                wf2_ref, bf2_ref, o_ref, acc):
    i = pl.program_id(0)
    deg = deg_ref[0, :, 0] + deg_ref[1, :, 0] + 1.0
    dis = lax.rsqrt(deg)
    agg = p_ref[0] + p_ref[1] + hs_ref[...]
    o2 = jnp.maximum(agg * dis[:, None] + b2_ref[...], 0.0)
    part = jnp.sum(o2, axis=0, keepdims=True)

    @pl.when(i == 0)
    def _():
        acc[...] = part

    @pl.when(i > 0)
    def _():
        acc[...] = acc[...] + part

    @pl.when(i == GRID - 1)
    def _():
        g = acc[...] * (1.0 / N_NODES)
        n1p = jnp.maximum(
            jnp.dot(n1_ref[...], wp_ref[...],
                    preferred_element_type=jnp.float32) + bp_ref[...], 0.0)
        n2p = jnp.maximum(
            jnp.dot(n2_ref[...], wp_ref[...],
                    preferred_element_type=jnp.float32) + bp_ref[...], 0.0)
        z = jnp.dot(g, w1g_ref[...], preferred_element_type=jnp.float32)
        z = z + jnp.dot(n1p, w1n1_ref[...], preferred_element_type=jnp.float32)
        z = z + jnp.dot(n2p, w1n2_ref[...], preferred_element_type=jnp.float32)
        z = z + jnp.dot(extra_ref[...], w1e_ref[...],
                        preferred_element_type=jnp.float32)
        z = jnp.maximum(z + bf1_ref[...], 0.0)
        o_ref[...] = jnp.dot(z, wf2_ref[...],
                             preferred_element_type=jnp.float32) + bf2_ref[...]


def _final(p, hs, degp, b2, n1, n2, wp, bp, extra,
           w1g, w1n1, w1n2, w1e, bf1, wf2, bf2):
    full = lambda s: pl.BlockSpec(s, lambda i: tuple(0 for _ in s))
    return pl.pallas_call(
        _final_body,
        grid=(GRID,),
        in_specs=[
            pl.BlockSpec((NC, BLK, D), lambda i: (0, i, 0)),
            pl.BlockSpec((BLK, D), lambda i: (i, 0)),
            pl.BlockSpec((NC, BLK, DEG_W), lambda i: (0, i, 0)),
            full((1, D)),
            full((1, D)),
            full((1, D)),
            full((D, D_PROJ)),
            full((1, D_PROJ)),
            full((1, 4)),
            full((D, D)),
            full((D_PROJ, D)),
            full((D_PROJ, D)),
            full((4, D)),
            full((1, D)),
            full((D, 1)),
            full((1, 1)),
        ],
        out_specs=pl.BlockSpec((1, 1), lambda i: (0, 0)),
        out_shape=jax.ShapeDtypeStruct((1, 1), jnp.float32),
        scratch_shapes=[pltpu.VMEM((1, D), jnp.float32)],
    )(p, hs, degp, b2, n1, n2, wp, bp, extra, w1g, w1n1, w1n2, w1e, bf1,
      wf2, bf2)


@jax.jit
def kernel(x, edge_index, action_one_hot, node1_emb, node2_emb, action_prob,
           W1, b1, W2, b2, Wp, bp, Wf1, bf1, Wf2, bf2):
    src2d = edge_index[0].reshape(NW, NCHUNK, CH)
    dst2d = edge_index[1].reshape(NW, NCHUNK, CH)
    ones8 = jnp.ones((CH, DEG_W), jnp.float32)
    zdeg = jnp.zeros((IO_ROWS, DEG_W), jnp.float32)
    zrow = jnp.zeros((IO_ROWS, D), jnp.float32)

    degp = _deg_kernel()(dst2d, ones8, zdeg)
    hs1 = _mm_scale(x, W1, degp)
    p1 = _edge_kernel()(hs1, src2d, dst2d, zrow)
    hs2 = _layer_mm(p1, hs1, degp, b1.reshape(1, D), W2)
    p2 = _edge_kernel()(hs2, src2d, dst2d, zrow)

    extra = jnp.concatenate([action_one_hot, action_prob], axis=1)
    out = _final(p2, hs2, degp, b2.reshape(1, D),
                 node1_emb, node2_emb, Wp, bp.reshape(1, D_PROJ), extra,
                 Wf1[:D], Wf1[D:D + D_PROJ], Wf1[D + D_PROJ:D + 2 * D_PROJ],
                 Wf1[D + 2 * D_PROJ:], bf1.reshape(1, D), Wf2,
                 bf2.reshape(1, 1))
    return out


# speedup vs baseline: 34.0763x; 1.7571x over previous
import functools

import jax
import jax.numpy as jnp
from jax import lax
from jax.experimental import pallas as pl
from jax.experimental.pallas import tpu as pltpu
from jax.experimental.pallas import tpu_sc as plsc

N_NODES = 10000
D = 128
N_EDGES = 320000
D_PROJ = 64

NC, NS = 2, 16
NW = NC * NS
E_PER_TILE = N_EDGES // NW
CH = 40
NCHUNK = E_PER_TILE // CH
NBUF = 5
ROWS_PER_TILE = N_NODES // NS
NS_IO = 10
IO_ROWS = N_NODES // NS_IO
DEG_W = 16

BLK = 1000
GRID = N_NODES // BLK


def _deg_body(dst_hbm, ones_hbm, zdeg_hbm, out_hbm, dst_v, ones_v, deg_sh):
    cid = lax.axis_index("c")
    sid = lax.axis_index("s")
    w = cid * NS + sid
    pltpu.sync_copy(dst_hbm.at[w], dst_v)
    pltpu.sync_copy(ones_hbm, ones_v)

    @pl.when(sid < NS_IO)
    def _():
        pltpu.sync_copy(zdeg_hbm, deg_sh.at[pl.ds(sid * IO_ROWS, IO_ROWS)])

    plsc.subcore_barrier()

    def body(j, carry):
        pltpu.sync_copy(ones_v, deg_sh.at[dst_v.at[j]], add=True)
        return carry

    lax.fori_loop(0, NCHUNK, body, 0)
    plsc.subcore_barrier()

    @pl.when(sid < NS_IO)
    def _():
        pltpu.sync_copy(deg_sh.at[pl.ds(sid * IO_ROWS, IO_ROWS)],
                        out_hbm.at[cid].at[pl.ds(sid * IO_ROWS, IO_ROWS)])


@functools.cache
def _sc_mesh():
    return plsc.VectorSubcoreMesh(
        core_axis_name="c", subcore_axis_name="s",
        num_cores=NC, num_subcores=NS)


@functools.cache
def _deg_kernel():
    return pl.kernel(
        _deg_body,
        out_type=jax.ShapeDtypeStruct((NC, N_NODES, DEG_W), jnp.float32),
        mesh=_sc_mesh(),
        compiler_params=pltpu.CompilerParams(use_tc_tiling_on_sc=False),
        scratch_types=[
            pltpu.VMEM((NCHUNK, CH), jnp.int32),
            pltpu.VMEM((CH, DEG_W), jnp.float32),
            pltpu.VMEM_SHARED((N_NODES, DEG_W), jnp.float32),
        ])


def _edge_body(hs_hbm, src_hbm, dst_hbm, zrow_hbm, out_hbm,
               src_v, dst_v, rows_v, acc_sh, gsem):
    cid = lax.axis_index("c")
    sid = lax.axis_index("s")
    w = cid * NS + sid
    pltpu.sync_copy(src_hbm.at[w], src_v)
    pltpu.sync_copy(dst_hbm.at[w], dst_v)

    @pl.when(sid < NS_IO)
    def _():
        pltpu.sync_copy(zrow_hbm, acc_sh.at[pl.ds(sid * IO_ROWS, IO_ROWS)])

    plsc.subcore_barrier()

    def _gather(c, b):
        pltpu.async_copy(hs_hbm.at[src_v.at[c]], rows_v.at[b], gsem.at[b])

    def _gather_wait(c, b):
        pltpu.make_async_copy(
            hs_hbm.at[src_v.at[c]], rows_v.at[b], gsem.at[b]).wait()

    def _scatter(c, b):
        pltpu.sync_copy(rows_v.at[b], acc_sh.at[dst_v.at[c]], add=True)

    for b in range(NBUF):
        _gather(b, b)

    def body(grp, carry):
        for b in range(NBUF):
            c = grp * NBUF + b
            _gather_wait(c, b)
            _scatter(c, b)
            c2 = c + NBUF

            @pl.when(c2 < NCHUNK)
            def _():
                _gather(c2, b)
        return carry

    lax.fori_loop(0, NCHUNK // NBUF, body, 0)
    plsc.subcore_barrier()

    @pl.when(sid < NS_IO)
    def _():
        pltpu.sync_copy(acc_sh.at[pl.ds(sid * IO_ROWS, IO_ROWS)],
                        out_hbm.at[cid].at[pl.ds(sid * IO_ROWS, IO_ROWS)])


@functools.cache
def _edge_kernel():
    return pl.kernel(
        _edge_body,
        out_type=jax.ShapeDtypeStruct((NC, N_NODES, D), jnp.float32),
        mesh=_sc_mesh(),
        compiler_params=pltpu.CompilerParams(use_tc_tiling_on_sc=False),
        scratch_types=[
            pltpu.VMEM((NCHUNK, CH), jnp.int32),
            pltpu.VMEM((NCHUNK, CH), jnp.int32),
            pltpu.VMEM((NBUF, CH, D), jnp.float32),
            pltpu.VMEM_SHARED((N_NODES, D), jnp.float32),
            pltpu.SemaphoreType.DMA((NBUF,)),
        ])


def _mm_scale_body(x_ref, w_ref, deg_ref, o_ref):
    deg = deg_ref[0, :, 0] + deg_ref[1, :, 0] + 1.0
    dis = lax.rsqrt(deg)
    h = jnp.dot(x_ref[...], w_ref[...], preferred_element_type=jnp.float32)
    o_ref[...] = h * dis[:, None]


def _mm_scale(x, w1, degp):
    return pl.pallas_call(
        _mm_scale_body,
        grid=(GRID,),
        in_specs=[
            pl.BlockSpec((BLK, D), lambda i: (i, 0)),
            pl.BlockSpec((D, D), lambda i: (0, 0)),
            pl.BlockSpec((NC, BLK, DEG_W), lambda i: (0, i, 0)),
        ],
        out_specs=pl.BlockSpec((BLK, D), lambda i: (i, 0)),
        out_shape=jax.ShapeDtypeStruct((N_NODES, D), jnp.float32),
    )(x, w1, degp)


def _layer_body(p_ref, hs_ref, deg_ref, b_ref, w_ref, o_ref):
    deg = deg_ref[0, :, 0] + deg_ref[1, :, 0] + 1.0
    dis = lax.rsqrt(deg)
    agg = p_ref[0] + p_ref[1] + hs_ref[...]
    o1 = jnp.maximum(agg * dis[:, None] + b_ref[...], 0.0)
    h2 = jnp.dot(o1, w_ref[...], preferred_element_type=jnp.float32)
    o_ref[...] = h2 * dis[:, None]


def _layer_mm(p, hs, degp, b, w2):
    return pl.pallas_call(
        _layer_body,
        grid=(GRID,),
        in_specs=[
            pl.BlockSpec((NC, BLK, D), lambda i: (0, i, 0)),
            pl.BlockSpec((BLK, D), lambda i: (i, 0)),
            pl.BlockSpec((NC, BLK, DEG_W), lambda i: (0, i, 0)),
            pl.BlockSpec((1, D), lambda i: (0, 0)),
            pl.BlockSpec((D, D), lambda i: (0, 0)),
        ],
        out_specs=pl.BlockSpec((BLK, D), lambda i: (i, 0)),
        out_shape=jax.ShapeDtypeStruct((N_NODES, D), jnp.float32),
    )(p, hs, degp, b, w2)


def _final_body(p_ref, hs_ref, deg_ref, b2_ref, n1_ref, n2_ref, wp_ref, bp_ref,
                extra_ref, w1g_ref, w1n1_ref, w1n2_ref, w1e_ref, bf1_ref,
                wf2_ref, bf2_ref, o_ref, acc):
    i = pl.program_id(0)
    deg = deg_ref[0, :, 0] + deg_ref[1, :, 0] + 1.0
    dis = lax.rsqrt(deg)
    agg = p_ref[0] + p_ref[1] + hs_ref[...]
    o2 = jnp.maximum(agg * dis[:, None] + b2_ref[...], 0.0)
    part = jnp.sum(o2, axis=0, keepdims=True)

    @pl.when(i == 0)
    def _():
        acc[...] = part

    @pl.when(i > 0)
    def _():
        acc[...] = acc[...] + part

    @pl.when(i == GRID - 1)
    def _():
        g = acc[...] * (1.0 / N_NODES)
        n1p = jnp.maximum(
            jnp.dot(n1_ref[...], wp_ref[...],
                    preferred_element_type=jnp.float32) + bp_ref[...], 0.0)
        n2p = jnp.maximum(
            jnp.dot(n2_ref[...], wp_ref[...],
                    preferred_element_type=jnp.float32) + bp_ref[...], 0.0)
        z = jnp.dot(g, w1g_ref[...], preferred_element_type=jnp.float32)
        z = z + jnp.dot(n1p, w1n1_ref[...], preferred_element_type=jnp.float32)
        z = z + jnp.dot(n2p, w1n2_ref[...], preferred_element_type=jnp.float32)
        z = z + jnp.dot(extra_ref[...], w1e_ref[...],
                        preferred_element_type=jnp.float32)
        z = jnp.maximum(z + bf1_ref[...], 0.0)
        o_ref[...] = jnp.dot(z, wf2_ref[...],
                             preferred_element_type=jnp.float32) + bf2_ref[...]


def _final(p, hs, degp, b2, n1, n2, wp, bp, extra,
           w1g, w1n1, w1n2, w1e, bf1, wf2, bf2):
    full = lambda s: pl.BlockSpec(s, lambda i: tuple(0 for _ in s))
    return pl.pallas_call(
        _final_body,
        grid=(GRID,),
        in_specs=[
            pl.BlockSpec((NC, BLK, D), lambda i: (0, i, 0)),
            pl.BlockSpec((BLK, D), lambda i: (i, 0)),
            pl.BlockSpec((NC, BLK, DEG_W), lambda i: (0, i, 0)),
            full((1, D)),
            full((1, D)),
            full((1, D)),
            full((D, D_PROJ)),
            full((1, D_PROJ)),
            full((1, 4)),
            full((D, D)),
            full((D_PROJ, D)),
            full((D_PROJ, D)),
            full((4, D)),
            full((1, D)),
            full((D, 1)),
            full((1, 1)),
        ],
        out_specs=pl.BlockSpec((1, 1), lambda i: (0, 0)),
        out_shape=jax.ShapeDtypeStruct((1, 1), jnp.float32),
        scratch_shapes=[pltpu.VMEM((1, D), jnp.float32)],
    )(p, hs, degp, b2, n1, n2, wp, bp, extra, w1g, w1n1, w1n2, w1e, bf1,
      wf2, bf2)


@jax.jit
def kernel(x, edge_index, action_one_hot, node1_emb, node2_emb, action_prob,
           W1, b1, W2, b2, Wp, bp, Wf1, bf1, Wf2, bf2):
    src2d = edge_index[0].reshape(NW, NCHUNK, CH)
    dst2d = edge_index[1].reshape(NW, NCHUNK, CH)
    ones8 = jnp.ones((CH, DEG_W), jnp.float32)
    zdeg = jnp.zeros((IO_ROWS, DEG_W), jnp.float32)
    zrow = jnp.zeros((IO_ROWS, D), jnp.float32)

    degp = _deg_kernel()(dst2d, ones8, zdeg)
    hs1 = _mm_scale(x, W1, degp)
    p1 = _edge_kernel()(hs1, src2d, dst2d, zrow)
    hs2 = _layer_mm(p1, hs1, degp, b1.reshape(1, D), W2)
    p2 = _edge_kernel()(hs2, src2d, dst2d, zrow)

    extra = jnp.concatenate([action_one_hot, action_prob], axis=1)
    out = _final(p2, hs2, degp, b2.reshape(1, D),
                 node1_emb, node2_emb, Wp, bp.reshape(1, D_PROJ), extra,
                 Wf1[:D], Wf1[D:D + D_PROJ], Wf1[D + D_PROJ:D + 2 * D_PROJ],
                 Wf1[D + 2 * D_PROJ:], bf1.reshape(1, D), Wf2,
                 bf2.reshape(1, 1))
    return out


# speedup vs baseline: 34.9867x; 1.0267x over previous
import functools

import jax
import jax.numpy as jnp
from jax import lax
from jax.experimental import pallas as pl
from jax.experimental.pallas import tpu as pltpu
from jax.experimental.pallas import tpu_sc as plsc

N_NODES = 10000
D = 128
N_EDGES = 320000
D_PROJ = 64

NC, NS = 2, 16
NW = NC * NS
E_PER_TILE = N_EDGES // NW
CH = 40
NCHUNK = E_PER_TILE // CH
NBUF = 5
ROWS_PER_TILE = N_NODES // NS
DCH = 125
DNCHUNK = E_PER_TILE // DCH
NS_IO = 10
IO_ROWS = N_NODES // NS_IO
DEG_W = 16

BLK = 1000
GRID = N_NODES // BLK


def _deg_body(dst_hbm, ones_hbm, zdeg_hbm, out_hbm, dst_v, ones_v, deg_sh):
    cid = lax.axis_index("c")
    sid = lax.axis_index("s")
    w = cid * NS + sid
    pltpu.sync_copy(dst_hbm.at[w], dst_v)
    pltpu.sync_copy(ones_hbm, ones_v)

    @pl.when(sid < NS_IO)
    def _():
        pltpu.sync_copy(zdeg_hbm, deg_sh.at[pl.ds(sid * IO_ROWS, IO_ROWS)])

    plsc.subcore_barrier()

    def body(j, carry):
        pltpu.sync_copy(ones_v, deg_sh.at[dst_v.at[j]], add=True)
        return carry

    lax.fori_loop(0, DNCHUNK, body, 0)
    plsc.subcore_barrier()

    @pl.when(sid < NS_IO)
    def _():
        pltpu.sync_copy(deg_sh.at[pl.ds(sid * IO_ROWS, IO_ROWS)],
                        out_hbm.at[cid].at[pl.ds(sid * IO_ROWS, IO_ROWS)])


@functools.cache
def _sc_mesh():
    return plsc.VectorSubcoreMesh(
        core_axis_name="c", subcore_axis_name="s",
        num_cores=NC, num_subcores=NS)


@functools.cache
def _deg_kernel():
    return pl.kernel(
        _deg_body,
        out_type=jax.ShapeDtypeStruct((NC, N_NODES, DEG_W), jnp.float32),
        mesh=_sc_mesh(),
        compiler_params=pltpu.CompilerParams(use_tc_tiling_on_sc=False),
        scratch_types=[
            pltpu.VMEM((DNCHUNK, DCH), jnp.int32),
            pltpu.VMEM((DCH, DEG_W), jnp.float32),
            pltpu.VMEM_SHARED((N_NODES, DEG_W), jnp.float32),
        ])


def _edge_body(hs_hbm, src_hbm, dst_hbm, zrow_hbm, out_hbm,
               src_v, dst_v, rows_v, acc_sh, gsem):
    cid = lax.axis_index("c")
    sid = lax.axis_index("s")
    w = cid * NS + sid
    pltpu.sync_copy(src_hbm.at[w], src_v)
    pltpu.sync_copy(dst_hbm.at[w], dst_v)

    @pl.when(sid < NS_IO)
    def _():
        pltpu.sync_copy(zrow_hbm, acc_sh.at[pl.ds(sid * IO_ROWS, IO_ROWS)])

    plsc.subcore_barrier()

    def _gather(c, b):
        pltpu.async_copy(hs_hbm.at[src_v.at[c]], rows_v.at[b], gsem.at[b])

    def _gather_wait(c, b):
        pltpu.make_async_copy(
            hs_hbm.at[src_v.at[c]], rows_v.at[b], gsem.at[b]).wait()

    def _scatter(c, b):
        pltpu.sync_copy(rows_v.at[b], acc_sh.at[dst_v.at[c]], add=True)

    for b in range(NBUF):
        _gather(b, b)

    def body(grp, carry):
        for b in range(NBUF):
            c = grp * NBUF + b
            _gather_wait(c, b)
            _scatter(c, b)
            c2 = c + NBUF

            @pl.when(c2 < NCHUNK)
            def _():
                _gather(c2, b)
        return carry

    lax.fori_loop(0, NCHUNK // NBUF, body, 0)
    plsc.subcore_barrier()

    @pl.when(sid < NS_IO)
    def _():
        pltpu.sync_copy(acc_sh.at[pl.ds(sid * IO_ROWS, IO_ROWS)],
                        out_hbm.at[cid].at[pl.ds(sid * IO_ROWS, IO_ROWS)])


@functools.cache
def _edge_kernel():
    return pl.kernel(
        _edge_body,
        out_type=jax.ShapeDtypeStruct((NC, N_NODES, D), jnp.float32),
        mesh=_sc_mesh(),
        compiler_params=pltpu.CompilerParams(use_tc_tiling_on_sc=False),
        scratch_types=[
            pltpu.VMEM((NCHUNK, CH), jnp.int32),
            pltpu.VMEM((NCHUNK, CH), jnp.int32),
            pltpu.VMEM((NBUF, CH, D), jnp.float32),
            pltpu.VMEM_SHARED((N_NODES, D), jnp.float32),
            pltpu.SemaphoreType.DMA((NBUF,)),
        ])


def _mm_body(x_ref, w_ref, o_ref):
    o_ref[...] = jnp.dot(x_ref[...], w_ref[...],
                         preferred_element_type=jnp.float32)


def _mm(x, w1):
    return pl.pallas_call(
        _mm_body,
        grid=(GRID,),
        in_specs=[
            pl.BlockSpec((BLK, D), lambda i: (i, 0)),
            pl.BlockSpec((D, D), lambda i: (0, 0)),
        ],
        out_specs=pl.BlockSpec((BLK, D), lambda i: (i, 0)),
        out_shape=jax.ShapeDtypeStruct((N_NODES, D), jnp.float32),
    )(x, w1)


def _scale_body(u_ref, deg_ref, o_ref):
    deg = deg_ref[0, :, 0] + deg_ref[1, :, 0] + 1.0
    dis = lax.rsqrt(deg)
    o_ref[...] = u_ref[...] * dis[:, None]


def _scale(u, degp):
    return pl.pallas_call(
        _scale_body,
        grid=(GRID,),
        in_specs=[
            pl.BlockSpec((BLK, D), lambda i: (i, 0)),
            pl.BlockSpec((NC, BLK, DEG_W), lambda i: (0, i, 0)),
        ],
        out_specs=pl.BlockSpec((BLK, D), lambda i: (i, 0)),
        out_shape=jax.ShapeDtypeStruct((N_NODES, D), jnp.float32),
    )(u, degp)


def _layer_body(p_ref, hs_ref, deg_ref, b_ref, w_ref, o_ref):
    deg = deg_ref[0, :, 0] + deg_ref[1, :, 0] + 1.0
    dis = lax.rsqrt(deg)
    agg = p_ref[0] + p_ref[1] + hs_ref[...]
    o1 = jnp.maximum(agg * dis[:, None] + b_ref[...], 0.0)
    h2 = jnp.dot(o1, w_ref[...], preferred_element_type=jnp.float32)
    o_ref[...] = h2 * dis[:, None]


def _layer_mm(p, hs, degp, b, w2):
    return pl.pallas_call(
        _layer_body,
        grid=(GRID,),
        in_specs=[
            pl.BlockSpec((NC, BLK, D), lambda i: (0, i, 0)),
            pl.BlockSpec((BLK, D), lambda i: (i, 0)),
            pl.BlockSpec((NC, BLK, DEG_W), lambda i: (0, i, 0)),
            pl.BlockSpec((1, D), lambda i: (0, 0)),
            pl.BlockSpec((D, D), lambda i: (0, 0)),
        ],
        out_specs=pl.BlockSpec((BLK, D), lambda i: (i, 0)),
        out_shape=jax.ShapeDtypeStruct((N_NODES, D), jnp.float32),
    )(p, hs, degp, b, w2)


def _final_body(p_ref, hs_ref, deg_ref, b2_ref, n1_ref, n2_ref, wp_ref, bp_ref,
                extra_ref, w1g_ref, w1n1_ref, w1n2_ref, w1e_ref, bf1_ref,
                wf2_ref, bf2_ref, o_ref, acc):
    i = pl.program_id(0)
    deg = deg_ref[0, :, 0] + deg_ref[1, :, 0] + 1.0
    dis = lax.rsqrt(deg)
    agg = p_ref[0] + p_ref[1] + hs_ref[...]
    o2 = jnp.maximum(agg * dis[:, None] + b2_ref[...], 0.0)
    part = jnp.sum(o2, axis=0, keepdims=True)

    @pl.when(i == 0)
    def _():
        acc[...] = part

    @pl.when(i > 0)
    def _():
        acc[...] = acc[...] + part

    @pl.when(i == GRID - 1)
    def _():
        g = acc[...] * (1.0 / N_NODES)
        n1p = jnp.maximum(
            jnp.dot(n1_ref[...], wp_ref[...],
                    preferred_element_type=jnp.float32) + bp_ref[...], 0.0)
        n2p = jnp.maximum(
            jnp.dot(n2_ref[...], wp_ref[...],
                    preferred_element_type=jnp.float32) + bp_ref[...], 0.0)
        z = jnp.dot(g, w1g_ref[...], preferred_element_type=jnp.float32)
        z = z + jnp.dot(n1p, w1n1_ref[...], preferred_element_type=jnp.float32)
        z = z + jnp.dot(n2p, w1n2_ref[...], preferred_element_type=jnp.float32)
        z = z + jnp.dot(extra_ref[...], w1e_ref[...],
                        preferred_element_type=jnp.float32)
        z = jnp.maximum(z + bf1_ref[...], 0.0)
        o_ref[...] = jnp.dot(z, wf2_ref[...],
                             preferred_element_type=jnp.float32) + bf2_ref[...]


def _final(p, hs, degp, b2, n1, n2, wp, bp, extra,
           w1g, w1n1, w1n2, w1e, bf1, wf2, bf2):
    full = lambda s: pl.BlockSpec(s, lambda i: tuple(0 for _ in s))
    return pl.pallas_call(
        _final_body,
        grid=(GRID,),
        in_specs=[
            pl.BlockSpec((NC, BLK, D), lambda i: (0, i, 0)),
            pl.BlockSpec((BLK, D), lambda i: (i, 0)),
            pl.BlockSpec((NC, BLK, DEG_W), lambda i: (0, i, 0)),
            full((1, D)),
            full((1, D)),
            full((1, D)),
            full((D, D_PROJ)),
            full((1, D_PROJ)),
            full((1, 4)),
            full((D, D)),
            full((D_PROJ, D)),
            full((D_PROJ, D)),
            full((4, D)),
            full((1, D)),
            full((D, 1)),
            full((1, 1)),
        ],
        out_specs=pl.BlockSpec((1, 1), lambda i: (0, 0)),
        out_shape=jax.ShapeDtypeStruct((1, 1), jnp.float32),
        scratch_shapes=[pltpu.VMEM((1, D), jnp.float32)],
    )(p, hs, degp, b2, n1, n2, wp, bp, extra, w1g, w1n1, w1n2, w1e, bf1,
      wf2, bf2)


@jax.jit
def kernel(x, edge_index, action_one_hot, node1_emb, node2_emb, action_prob,
           W1, b1, W2, b2, Wp, bp, Wf1, bf1, Wf2, bf2):
    src2d = edge_index[0].reshape(NW, NCHUNK, CH)
    dst2d = edge_index[1].reshape(NW, NCHUNK, CH)
    ones8 = jnp.ones((DCH, DEG_W), jnp.float32)
    zdeg = jnp.zeros((IO_ROWS, DEG_W), jnp.float32)
    zrow = jnp.zeros((IO_ROWS, D), jnp.float32)

    dst2d_deg = edge_index[1].reshape(NW, DNCHUNK, DCH)
    degp = _deg_kernel()(dst2d_deg, ones8, zdeg)
    u1 = _mm(x, W1)
    hs1 = _scale(u1, degp)
    p1 = _edge_kernel()(hs1, src2d, dst2d, zrow)
    hs2 = _layer_mm(p1, hs1, degp, b1.reshape(1, D), W2)
    p2 = _edge_kernel()(hs2, src2d, dst2d, zrow)

    extra = jnp.concatenate([action_one_hot, action_prob], axis=1)
    out = _final(p2, hs2, degp, b2.reshape(1, D),
                 node1_emb, node2_emb, Wp, bp.reshape(1, D_PROJ), extra,
                 Wf1[:D], Wf1[D:D + D_PROJ], Wf1[D + D_PROJ:D + 2 * D_PROJ],
                 Wf1[D + 2 * D_PROJ:], bf1.reshape(1, D), Wf2,
                 bf2.reshape(1, 1))
    return out


# speedup vs baseline: 35.0446x; 1.0017x over previous
import functools

import jax
import jax.numpy as jnp
from jax import lax
from jax.experimental import pallas as pl
from jax.experimental.pallas import tpu as pltpu
from jax.experimental.pallas import tpu_sc as plsc

N_NODES = 10000
D = 128
N_EDGES = 320000
D_PROJ = 64

NC, NS = 2, 16
NW = NC * NS
E_PER_TILE = N_EDGES // NW
CH = 40
NCHUNK = E_PER_TILE // CH
NBUF = 5
ROWS_PER_TILE = N_NODES // NS
DCH = 125
DNCHUNK = E_PER_TILE // DCH
NS_IO = 10
IO_ROWS = N_NODES // NS_IO
DEG_W = 16

BLK = 1000
GRID = N_NODES // BLK


def _deg_body(dst_hbm, ones_hbm, zdeg_hbm, out_hbm, dst_v, ones_v, deg_sh):
    cid = lax.axis_index("c")
    sid = lax.axis_index("s")
    w = cid * NS + sid
    pltpu.sync_copy(dst_hbm.at[w], dst_v)
    pltpu.sync_copy(ones_hbm, ones_v)

    @pl.when(sid < NS_IO)
    def _():
        pltpu.sync_copy(zdeg_hbm, deg_sh.at[pl.ds(sid * IO_ROWS, IO_ROWS)])

    plsc.subcore_barrier()

    def body(j, carry):
        pltpu.sync_copy(ones_v, deg_sh.at[dst_v.at[j]], add=True)
        return carry

    lax.fori_loop(0, DNCHUNK, body, 0)
    plsc.subcore_barrier()

    @pl.when(sid < NS_IO)
    def _():
        pltpu.sync_copy(deg_sh.at[pl.ds(sid * IO_ROWS, IO_ROWS)],
                        out_hbm.at[cid].at[pl.ds(sid * IO_ROWS, IO_ROWS)])


@functools.cache
def _sc_mesh():
    return plsc.VectorSubcoreMesh(
        core_axis_name="c", subcore_axis_name="s",
        num_cores=NC, num_subcores=NS)


@functools.cache
def _deg_kernel():
    return pl.kernel(
        _deg_body,
        out_type=jax.ShapeDtypeStruct((NC, N_NODES, DEG_W), jnp.float32),
        mesh=_sc_mesh(),
        compiler_params=pltpu.CompilerParams(use_tc_tiling_on_sc=False),
        scratch_types=[
            pltpu.VMEM((DNCHUNK, DCH), jnp.int32),
            pltpu.VMEM((DCH, DEG_W), jnp.float32),
            pltpu.VMEM_SHARED((N_NODES, DEG_W), jnp.float32),
        ])


def _edge_body(hs_hbm, src_hbm, dst_hbm, zrow_hbm, out_hbm,
               src_v, dst_v, rows_v, acc_sh, gsem):
    cid = lax.axis_index("c")
    sid = lax.axis_index("s")
    w = cid * NS + sid
    pltpu.sync_copy(src_hbm.at[w], src_v)
    pltpu.sync_copy(dst_hbm.at[w], dst_v)

    @pl.when(sid < NS_IO)
    def _():
        pltpu.sync_copy(zrow_hbm, acc_sh.at[pl.ds(sid * IO_ROWS, IO_ROWS)])

    plsc.subcore_barrier()

    def _gather(c, b):
        pltpu.async_copy(hs_hbm.at[src_v.at[c]], rows_v.at[b], gsem.at[b])

    def _gather_wait(c, b):
        pltpu.make_async_copy(
            hs_hbm.at[src_v.at[c]], rows_v.at[b], gsem.at[b]).wait()

    def _scatter(c, b):
        pltpu.sync_copy(rows_v.at[b], acc_sh.at[dst_v.at[c]], add=True)

    for b in range(NBUF):
        _gather(b, b)

    def body(grp, carry):
        for b in range(NBUF):
            c = grp * NBUF + b
            _gather_wait(c, b)
            _scatter(c, b)
            c2 = c + NBUF

            @pl.when(c2 < NCHUNK)
            def _():
                _gather(c2, b)
        return carry

    lax.fori_loop(0, NCHUNK // NBUF, body, 0)
    plsc.subcore_barrier()

    @pl.when(sid < NS_IO)
    def _():
        pltpu.sync_copy(acc_sh.at[pl.ds(sid * IO_ROWS, IO_ROWS)],
                        out_hbm.at[cid].at[pl.ds(sid * IO_ROWS, IO_ROWS)])


@functools.cache
def _edge_kernel():
    return pl.kernel(
        _edge_body,
        out_type=jax.ShapeDtypeStruct((NC, N_NODES, D), jnp.float32),
        mesh=_sc_mesh(),
        compiler_params=pltpu.CompilerParams(use_tc_tiling_on_sc=False),
        scratch_types=[
            pltpu.VMEM((NCHUNK, CH), jnp.int32),
            pltpu.VMEM((NCHUNK, CH), jnp.int32),
            pltpu.VMEM((NBUF, CH, D), jnp.float32),
            pltpu.VMEM_SHARED((N_NODES, D), jnp.float32),
            pltpu.SemaphoreType.DMA((NBUF,)),
        ])


def _mm_scale_body(x_ref, w_ref, deg_ref, o_ref):
    deg = deg_ref[0, :, 0] + deg_ref[1, :, 0] + 1.0
    dis = lax.rsqrt(deg)
    h = jnp.dot(x_ref[...] * dis[:, None], w_ref[...],
                preferred_element_type=jnp.float32)
    o_ref[...] = h


def _mm_scale(x, w1, degp):
    return pl.pallas_call(
        _mm_scale_body,
        grid=(GRID,),
        in_specs=[
            pl.BlockSpec((BLK, D), lambda i: (i, 0)),
            pl.BlockSpec((D, D), lambda i: (0, 0)),
            pl.BlockSpec((NC, BLK, DEG_W), lambda i: (0, i, 0)),
        ],
        out_specs=pl.BlockSpec((BLK, D), lambda i: (i, 0)),
        out_shape=jax.ShapeDtypeStruct((N_NODES, D), jnp.float32),
    )(x, w1, degp)


def _layer_body(p_ref, hs_ref, deg_ref, b_ref, w_ref, o_ref):
    deg = deg_ref[0, :, 0] + deg_ref[1, :, 0] + 1.0
    dis = lax.rsqrt(deg)
    agg = p_ref[0] + p_ref[1] + hs_ref[...]
    o1 = jnp.maximum(agg * dis[:, None] + b_ref[...], 0.0)
    h2 = jnp.dot(o1, w_ref[...], preferred_element_type=jnp.float32)
    o_ref[...] = h2 * dis[:, None]


def _layer_mm(p, hs, degp, b, w2):
    return pl.pallas_call(
        _layer_body,
        grid=(GRID,),
        in_specs=[
            pl.BlockSpec((NC, BLK, D), lambda i: (0, i, 0)),
            pl.BlockSpec((BLK, D), lambda i: (i, 0)),
            pl.BlockSpec((NC, BLK, DEG_W), lambda i: (0, i, 0)),
            pl.BlockSpec((1, D), lambda i: (0, 0)),
            pl.BlockSpec((D, D), lambda i: (0, 0)),
        ],
        out_specs=pl.BlockSpec((BLK, D), lambda i: (i, 0)),
        out_shape=jax.ShapeDtypeStruct((N_NODES, D), jnp.float32),
    )(p, hs, degp, b, w2)


def _final_body(p_ref, hs_ref, deg_ref, b2_ref, n1_ref, n2_ref, wp_ref, bp_ref,
                extra_ref, w1g_ref, w1n1_ref, w1n2_ref, w1e_ref, bf1_ref,
                wf2_ref, bf2_ref, o_ref, acc):
    i = pl.program_id(0)
    deg = deg_ref[0, :, 0] + deg_ref[1, :, 0] + 1.0
    dis = lax.rsqrt(deg)
    agg = p_ref[0] + p_ref[1] + hs_ref[...]
    o2 = jnp.maximum(agg * dis[:, None] + b2_ref[...], 0.0)
    part = jnp.sum(o2, axis=0, keepdims=True)

    @pl.when(i == 0)
    def _():
        acc[...] = part

    @pl.when(i > 0)
    def _():
        acc[...] = acc[...] + part

    @pl.when(i == GRID - 1)
    def _():
        g = acc[...] * (1.0 / N_NODES)
        n1p = jnp.maximum(
            jnp.dot(n1_ref[...], wp_ref[...],
                    preferred_element_type=jnp.float32) + bp_ref[...], 0.0)
        n2p = jnp.maximum(
            jnp.dot(n2_ref[...], wp_ref[...],
                    preferred_element_type=jnp.float32) + bp_ref[...], 0.0)
        z = jnp.dot(g, w1g_ref[...], preferred_element_type=jnp.float32)
        z = z + jnp.dot(n1p, w1n1_ref[...], preferred_element_type=jnp.float32)
        z = z + jnp.dot(n2p, w1n2_ref[...], preferred_element_type=jnp.float32)
        z = z + jnp.dot(extra_ref[...], w1e_ref[...],
                        preferred_element_type=jnp.float32)
        z = jnp.maximum(z + bf1_ref[...], 0.0)
        o_ref[...] = jnp.dot(z, wf2_ref[...],
                             preferred_element_type=jnp.float32) + bf2_ref[...]


def _final(p, hs, degp, b2, n1, n2, wp, bp, extra,
           w1g, w1n1, w1n2, w1e, bf1, wf2, bf2):
    full = lambda s: pl.BlockSpec(s, lambda i: tuple(0 for _ in s))
    return pl.pallas_call(
        _final_body,
        grid=(GRID,),
        in_specs=[
            pl.BlockSpec((NC, BLK, D), lambda i: (0, i, 0)),
            pl.BlockSpec((BLK, D), lambda i: (i, 0)),
            pl.BlockSpec((NC, BLK, DEG_W), lambda i: (0, i, 0)),
            full((1, D)),
            full((1, D)),
            full((1, D)),
            full((D, D_PROJ)),
            full((1, D_PROJ)),
            full((1, 4)),
            full((D, D)),
            full((D_PROJ, D)),
            full((D_PROJ, D)),
            full((4, D)),
            full((1, D)),
            full((D, 1)),
            full((1, 1)),
        ],
        out_specs=pl.BlockSpec((1, 1), lambda i: (0, 0)),
        out_shape=jax.ShapeDtypeStruct((1, 1), jnp.float32),
        scratch_shapes=[pltpu.VMEM((1, D), jnp.float32)],
    )(p, hs, degp, b2, n1, n2, wp, bp, extra, w1g, w1n1, w1n2, w1e, bf1,
      wf2, bf2)


@jax.jit
def kernel(x, edge_index, action_one_hot, node1_emb, node2_emb, action_prob,
           W1, b1, W2, b2, Wp, bp, Wf1, bf1, Wf2, bf2):
    src2d = edge_index[0].reshape(NW, NCHUNK, CH)
    dst2d = edge_index[1].reshape(NW, NCHUNK, CH)
    ones8 = jnp.ones((DCH, DEG_W), jnp.float32)
    zdeg = jnp.zeros((IO_ROWS, DEG_W), jnp.float32)
    zrow = jnp.zeros((IO_ROWS, D), jnp.float32)

    dst2d_deg = edge_index[1].reshape(NW, DNCHUNK, DCH)
    degp = _deg_kernel()(dst2d_deg, ones8, zdeg)
    hs1 = _mm_scale(x, W1, degp)
    p1 = _edge_kernel()(hs1, src2d, dst2d, zrow)
    hs2 = _layer_mm(p1, hs1, degp, b1.reshape(1, D), W2)
    p2 = _edge_kernel()(hs2, src2d, dst2d, zrow)

    extra = jnp.concatenate([action_one_hot, action_prob], axis=1)
    out = _final(p2, hs2, degp, b2.reshape(1, D),
                 node1_emb, node2_emb, Wp, bp.reshape(1, D_PROJ), extra,
                 Wf1[:D], Wf1[D:D + D_PROJ], Wf1[D + D_PROJ:D + 2 * D_PROJ],
                 Wf1[D + 2 * D_PROJ:], bf1.reshape(1, D), Wf2,
                 bf2.reshape(1, 1))
    return out


# speedup vs baseline: 35.4176x; 1.0106x over previous
import functools

import jax
import jax.numpy as jnp
from jax import lax
from jax.experimental import pallas as pl
from jax.experimental.pallas import tpu as pltpu
from jax.experimental.pallas import tpu_sc as plsc

N_NODES = 10000
D = 128
N_EDGES = 320000
D_PROJ = 64

NC, NS = 2, 16
NW = NC * NS
E_PER_TILE = N_EDGES // NW
CH = 40
NCHUNK = E_PER_TILE // CH
NBUF = 5
ROWS_PER_TILE = N_NODES // NS
DCH = 125
DNCHUNK = E_PER_TILE // DCH
NS_IO = 10
IO_ROWS = N_NODES // NS_IO
DEG_W = 16

BLK = 1000
GRID = N_NODES // BLK


def _deg_body(dst_hbm, ones_hbm, zdeg_hbm, out_hbm, dst_v, ones_v, deg_sh):
    cid = lax.axis_index("c")
    sid = lax.axis_index("s")
    w = cid * NS + sid
    pltpu.sync_copy(dst_hbm.at[w], dst_v)
    pltpu.sync_copy(ones_hbm, ones_v)

    @pl.when(sid < NS_IO)
    def _():
        pltpu.sync_copy(zdeg_hbm, deg_sh.at[pl.ds(sid * IO_ROWS, IO_ROWS)])

    plsc.subcore_barrier()

    def body(j, carry):
        pltpu.sync_copy(ones_v, deg_sh.at[dst_v.at[j]], add=True)
        return carry

    lax.fori_loop(0, DNCHUNK, body, 0)
    plsc.subcore_barrier()

    @pl.when(sid < NS_IO)
    def _():
        pltpu.sync_copy(deg_sh.at[pl.ds(sid * IO_ROWS, IO_ROWS)],
                        out_hbm.at[cid].at[pl.ds(sid * IO_ROWS, IO_ROWS)])


@functools.cache
def _sc_mesh():
    return plsc.VectorSubcoreMesh(
        core_axis_name="c", subcore_axis_name="s",
        num_cores=NC, num_subcores=NS)


@functools.cache
def _deg_kernel():
    return pl.kernel(
        _deg_body,
        out_type=jax.ShapeDtypeStruct((NC, N_NODES, DEG_W), jnp.float32),
        mesh=_sc_mesh(),
        compiler_params=pltpu.CompilerParams(use_tc_tiling_on_sc=False),
        scratch_types=[
            pltpu.VMEM((DNCHUNK, DCH), jnp.int32),
            pltpu.VMEM((DCH, DEG_W), jnp.float32),
            pltpu.VMEM_SHARED((N_NODES, DEG_W), jnp.float32),
        ])


def _edge_body(hs_hbm, src_hbm, dst_hbm, zrow_hbm, out_hbm,
               src_v, dst_v, rows_v, acc_sh, gsem):
    cid = lax.axis_index("c")
    sid = lax.axis_index("s")
    w = cid * NS + sid
    pltpu.sync_copy(src_hbm.at[w], src_v)
    pltpu.sync_copy(dst_hbm.at[w], dst_v)

    def _gather(c, b):
        pltpu.async_copy(hs_hbm.at[src_v.at[c]], rows_v.at[b], gsem.at[b])

    def _gather_wait(c, b):
        pltpu.make_async_copy(
            hs_hbm.at[src_v.at[c]], rows_v.at[b], gsem.at[b]).wait()

    def _scatter(c, b):
        pltpu.sync_copy(rows_v.at[b], acc_sh.at[dst_v.at[c]], add=True)

    for b in range(NBUF):
        _gather(b, b)

    @pl.when(sid < NS_IO)
    def _():
        pltpu.sync_copy(zrow_hbm, acc_sh.at[pl.ds(sid * IO_ROWS, IO_ROWS)])

    plsc.subcore_barrier()

    def body(grp, carry):
        for b in range(NBUF):
            c = grp * NBUF + b
            _gather_wait(c, b)
            _scatter(c, b)
            c2 = c + NBUF

            @pl.when(c2 < NCHUNK)
            def _():
                _gather(c2, b)
        return carry

    lax.fori_loop(0, NCHUNK // NBUF, body, 0)
    plsc.subcore_barrier()

    @pl.when(sid < NS_IO)
    def _():
        pltpu.sync_copy(acc_sh.at[pl.ds(sid * IO_ROWS, IO_ROWS)],
                        out_hbm.at[cid].at[pl.ds(sid * IO_ROWS, IO_ROWS)])


@functools.cache
def _edge_kernel():
    return pl.kernel(
        _edge_body,
        out_type=jax.ShapeDtypeStruct((NC, N_NODES, D), jnp.float32),
        mesh=_sc_mesh(),
        compiler_params=pltpu.CompilerParams(use_tc_tiling_on_sc=False),
        scratch_types=[
            pltpu.VMEM((NCHUNK, CH), jnp.int32),
            pltpu.VMEM((NCHUNK, CH), jnp.int32),
            pltpu.VMEM((NBUF, CH, D), jnp.float32),
            pltpu.VMEM_SHARED((N_NODES, D), jnp.float32),
            pltpu.SemaphoreType.DMA((NBUF,)),
        ])


def _mm_scale_body(x_ref, w_ref, deg_ref, o_ref):
    deg = deg_ref[0, :, 0] + deg_ref[1, :, 0] + 1.0
    dis = lax.rsqrt(deg)
    h = jnp.dot(x_ref[...] * dis[:, None], w_ref[...],
                preferred_element_type=jnp.float32)
    o_ref[...] = h


def _mm_scale(x, w1, degp):
    return pl.pallas_call(
        _mm_scale_body,
        grid=(GRID,),
        in_specs=[
            pl.BlockSpec((BLK, D), lambda i: (i, 0)),
            pl.BlockSpec((D, D), lambda i: (0, 0)),
            pl.BlockSpec((NC, BLK, DEG_W), lambda i: (0, i, 0)),
        ],
        out_specs=pl.BlockSpec((BLK, D), lambda i: (i, 0)),
        out_shape=jax.ShapeDtypeStruct((N_NODES, D), jnp.float32),
    )(x, w1, degp)


def _layer_body(p_ref, hs_ref, deg_ref, b_ref, w_ref, o_ref):
    deg = deg_ref[0, :, 0] + deg_ref[1, :, 0] + 1.0
    dis = lax.rsqrt(deg)
    agg = p_ref[0] + p_ref[1] + hs_ref[...]
    o1 = jnp.maximum(agg * dis[:, None] + b_ref[...], 0.0)
    h2 = jnp.dot(o1, w_ref[...], preferred_element_type=jnp.float32)
    o_ref[...] = h2 * dis[:, None]


def _layer_mm(p, hs, degp, b, w2):
    return pl.pallas_call(
        _layer_body,
        grid=(GRID,),
        in_specs=[
            pl.BlockSpec((NC, BLK, D), lambda i: (0, i, 0)),
            pl.BlockSpec((BLK, D), lambda i: (i, 0)),
            pl.BlockSpec((NC, BLK, DEG_W), lambda i: (0, i, 0)),
            pl.BlockSpec((1, D), lambda i: (0, 0)),
            pl.BlockSpec((D, D), lambda i: (0, 0)),
        ],
        out_specs=pl.BlockSpec((BLK, D), lambda i: (i, 0)),
        out_shape=jax.ShapeDtypeStruct((N_NODES, D), jnp.float32),
    )(p, hs, degp, b, w2)


def _final_body(p_ref, hs_ref, deg_ref, b2_ref, n1_ref, n2_ref, wp_ref, bp_ref,
                extra_ref, w1g_ref, w1n1_ref, w1n2_ref, w1e_ref, bf1_ref,
                wf2_ref, bf2_ref, o_ref, acc):
    i = pl.program_id(0)
    deg = deg_ref[0, :, 0] + deg_ref[1, :, 0] + 1.0
    dis = lax.rsqrt(deg)
    agg = p_ref[0] + p_ref[1] + hs_ref[...]
    o2 = jnp.maximum(agg * dis[:, None] + b2_ref[...], 0.0)
    part = jnp.sum(o2, axis=0, keepdims=True)

    @pl.when(i == 0)
    def _():
        acc[...] = part

    @pl.when(i > 0)
    def _():
        acc[...] = acc[...] + part

    @pl.when(i == GRID - 1)
    def _():
        g = acc[...] * (1.0 / N_NODES)
        n1p = jnp.maximum(
            jnp.dot(n1_ref[...], wp_ref[...],
                    preferred_element_type=jnp.float32) + bp_ref[...], 0.0)
        n2p = jnp.maximum(
            jnp.dot(n2_ref[...], wp_ref[...],
                    preferred_element_type=jnp.float32) + bp_ref[...], 0.0)
        z = jnp.dot(g, w1g_ref[...], preferred_element_type=jnp.float32)
        z = z + jnp.dot(n1p, w1n1_ref[...], preferred_element_type=jnp.float32)
        z = z + jnp.dot(n2p, w1n2_ref[...], preferred_element_type=jnp.float32)
        z = z + jnp.dot(extra_ref[...], w1e_ref[...],
                        preferred_element_type=jnp.float32)
        z = jnp.maximum(z + bf1_ref[...], 0.0)
        o_ref[...] = jnp.dot(z, wf2_ref[...],
                             preferred_element_type=jnp.float32) + bf2_ref[...]


def _final(p, hs, degp, b2, n1, n2, wp, bp, extra,
           w1g, w1n1, w1n2, w1e, bf1, wf2, bf2):
    full = lambda s: pl.BlockSpec(s, lambda i: tuple(0 for _ in s))
    return pl.pallas_call(
        _final_body,
        grid=(GRID,),
        in_specs=[
            pl.BlockSpec((NC, BLK, D), lambda i: (0, i, 0)),
            pl.BlockSpec((BLK, D), lambda i: (i, 0)),
            pl.BlockSpec((NC, BLK, DEG_W), lambda i: (0, i, 0)),
            full((1, D)),
            full((1, D)),
            full((1, D)),
            full((D, D_PROJ)),
            full((1, D_PROJ)),
            full((1, 4)),
            full((D, D)),
            full((D_PROJ, D)),
            full((D_PROJ, D)),
            full((4, D)),
            full((1, D)),
            full((D, 1)),
            full((1, 1)),
        ],
        out_specs=pl.BlockSpec((1, 1), lambda i: (0, 0)),
        out_shape=jax.ShapeDtypeStruct((1, 1), jnp.float32),
        scratch_shapes=[pltpu.VMEM((1, D), jnp.float32)],
    )(p, hs, degp, b2, n1, n2, wp, bp, extra, w1g, w1n1, w1n2, w1e, bf1,
      wf2, bf2)


@jax.jit
def kernel(x, edge_index, action_one_hot, node1_emb, node2_emb, action_prob,
           W1, b1, W2, b2, Wp, bp, Wf1, bf1, Wf2, bf2):
    src2d = edge_index[0].reshape(NW, NCHUNK, CH)
    dst2d = edge_index[1].reshape(NW, NCHUNK, CH)
    ones8 = jnp.ones((DCH, DEG_W), jnp.float32)
    zdeg = jnp.zeros((IO_ROWS, DEG_W), jnp.float32)
    zrow = jnp.zeros((IO_ROWS, D), jnp.float32)

    dst2d_deg = edge_index[1].reshape(NW, DNCHUNK, DCH)
    degp = _deg_kernel()(dst2d_deg, ones8, zdeg)
    hs1 = _mm_scale(x, W1, degp)
    p1 = _edge_kernel()(hs1, src2d, dst2d, zrow)
    hs2 = _layer_mm(p1, hs1, degp, b1.reshape(1, D), W2)
    p2 = _edge_kernel()(hs2, src2d, dst2d, zrow)

    extra = jnp.concatenate([action_one_hot, action_prob], axis=1)
    out = _final(p2, hs2, degp, b2.reshape(1, D),
                 node1_emb, node2_emb, Wp, bp.reshape(1, D_PROJ), extra,
                 Wf1[:D], Wf1[D:D + D_PROJ], Wf1[D + D_PROJ:D + 2 * D_PROJ],
                 Wf1[D + 2 * D_PROJ:], bf1.reshape(1, D), Wf2,
                 bf2.reshape(1, 1))
    return out
